# baseline (speedup 1.0000x reference)
import jax
import jax.numpy as jnp
from jax.experimental import pallas as pl
from jax.experimental.pallas import tpu as pltpu

_B_BLK = 256
_PAD = 128


def _prep_kernel(mv_ref, wo_ref, w2_ref, g1_ref, be1_ref, b2_ref,
                 waug_ref, consts_ref):
    kk = mv_ref.shape[0]
    vv = mv_ref.shape[1]
    hh = wo_ref.shape[1] // vv
    dd = wo_ref.shape[0]
    oo = w2_ref.shape[0]
    vmean = jnp.sum(mv_ref[...], axis=0, keepdims=True) / kk
    c_hv = jnp.concatenate([vmean] * hh, axis=1)
    c_vec = jax.lax.dot_general(
        c_hv, wo_ref[...], dimension_numbers=(((1,), (1,)), ((), ())),
        preferred_element_type=jnp.float32)
    w2g = w2_ref[...] * g1_ref[...]
    s_row = jax.lax.dot_general(
        g1_ref[...], w2_ref[...], dimension_numbers=(((1,), (1,)), ((), ())),
        preferred_element_type=jnp.float32)
    t_row = jax.lax.dot_general(
        be1_ref[...], w2_ref[...], dimension_numbers=(((1,), (1,)), ((), ())),
        preferred_element_type=jnp.float32) + b2_ref[...]
    cp_row = jax.lax.dot_general(
        c_vec, w2g, dimension_numbers=(((1,), (1,)), ((), ())),
        preferred_element_type=jnp.float32)
    sum_c = jnp.sum(c_vec, axis=1, keepdims=True)
    sum_c2 = jnp.sum(c_vec * c_vec, axis=1, keepdims=True)

    waug_ref[0:oo, :] = w2g.astype(jnp.bfloat16)
    waug_ref[oo:oo + 1, :] = jnp.ones((1, dd), dtype=jnp.bfloat16)
    waug_ref[oo + 1:oo + 2, :] = c_vec.astype(jnp.bfloat16)
    waug_ref[oo + 2:, :] = jnp.zeros((_PAD - oo - 2, dd), dtype=jnp.bfloat16)

    row = jnp.zeros((1, _PAD), dtype=jnp.float32)
    consts_ref[0:1, 0:oo] = cp_row
    consts_ref[0:1, oo:] = jnp.zeros((1, _PAD - oo), dtype=jnp.float32)
    consts_ref[1:2, 0:oo] = s_row
    consts_ref[1:2, oo:] = jnp.zeros((1, _PAD - oo), dtype=jnp.float32)
    consts_ref[2:3, 0:oo] = t_row
    consts_ref[2:3, oo:] = jnp.zeros((1, _PAD - oo), dtype=jnp.float32)
    consts_ref[3:4, :] = row + sum_c
    consts_ref[4:5, :] = row + sum_c2
    consts_ref[5:8, :] = jnp.zeros((3, _PAD), dtype=jnp.float32)


def _main_kernel(x_ref, w1_ref, b1_ref, waug_ref, consts_ref,
                 g2_ref, be2_ref, out_ref):
    dd = w1_ref.shape[0]
    oo = out_ref.shape[1]
    h = jax.lax.dot_general(
        x_ref[...], w1_ref[...],
        dimension_numbers=(((1,), (1,)), ((), ())),
        preferred_element_type=jnp.float32)
    h = jnp.maximum(h + b1_ref[...], 0.0)
    hsq = jnp.sum(h * h, axis=1, keepdims=True)
    hb = h.astype(jnp.bfloat16)
    p = jax.lax.dot_general(
        hb, waug_ref[...], dimension_numbers=(((1,), (1,)), ((), ())),
        preferred_element_type=jnp.float32)

    sum_c = consts_ref[3:4, 0:1]
    sum_c2 = consts_ref[4:5, 0:1]
    mean1 = (p[:, oo:oo + 1] + sum_c) / dd
    e2 = (hsq + 2.0 * p[:, oo + 1:oo + 2] + sum_c2) / dd
    var1 = e2 - mean1 * mean1
    rstd1 = 1.0 / jnp.sqrt(var1 + 1e-5)
    x2 = ((p[:, 0:oo] + consts_ref[0:1, 0:oo] - mean1 * consts_ref[1:2, 0:oo])
          * rstd1 + consts_ref[2:3, 0:oo])

    mean2 = jnp.mean(x2, axis=1, keepdims=True)
    cen2 = x2 - mean2
    var2 = jnp.mean(cen2 * cen2, axis=1, keepdims=True)
    y = cen2 / jnp.sqrt(var2 + 1e-5) * g2_ref[...] + be2_ref[...]
    out_ref[...] = jax.nn.sigmoid(y)


def kernel(X, W1, b1, mem_keys, mem_values, Wq, Wo, ln1_g, ln1_b,
           W2, b2, ln2_g, ln2_b):
    del mem_keys, Wq
    B, D = X.shape
    O = W2.shape[0]

    waug, consts = pl.pallas_call(
        _prep_kernel,
        in_specs=[pl.BlockSpec(mem_values.shape, lambda: (0, 0)),
                  pl.BlockSpec(Wo.shape, lambda: (0, 0)),
                  pl.BlockSpec(W2.shape, lambda: (0, 0)),
                  pl.BlockSpec((1, D), lambda: (0, 0)),
                  pl.BlockSpec((1, D), lambda: (0, 0)),
                  pl.BlockSpec((1, O), lambda: (0, 0))],
        out_specs=[pl.BlockSpec((_PAD, D), lambda: (0, 0)),
                   pl.BlockSpec((8, _PAD), lambda: (0, 0))],
        out_shape=[jax.ShapeDtypeStruct((_PAD, D), jnp.bfloat16),
                   jax.ShapeDtypeStruct((8, _PAD), jnp.float32)],
    )(mem_values, Wo, W2, ln1_g.reshape(1, D), ln1_b.reshape(1, D),
      b2.reshape(1, O))

    grid = (B // _B_BLK,)

    def rows(i):
        return (i, 0)

    def whole(i):
        return (0, 0)

    return pl.pallas_call(
        _main_kernel,
        grid=grid,
        in_specs=[pl.BlockSpec((_B_BLK, D), rows),
                  pl.BlockSpec((D, D), whole),
                  pl.BlockSpec((1, D), whole),
                  pl.BlockSpec((_PAD, D), whole),
                  pl.BlockSpec((8, _PAD), whole),
                  pl.BlockSpec((1, O), whole),
                  pl.BlockSpec((1, O), whole)],
        out_specs=pl.BlockSpec((_B_BLK, O), rows),
        out_shape=jax.ShapeDtypeStruct((B, O), jnp.float32),
    )(X.astype(jnp.bfloat16), W1.astype(jnp.bfloat16), b1.reshape(1, D),
      waug, consts, ln2_g.reshape(1, O), ln2_b.reshape(1, O))


# speedup vs baseline: 1.3517x; 1.3517x over previous
import jax
import jax.numpy as jnp
from jax.experimental import pallas as pl

_B_BLK = 256
_PAD = 128


def _fused_kernel(x_ref, w1_ref, b1_ref, mv_ref, wo_ref, g1_ref, be1_ref,
                  w2_ref, b2_ref, g2_ref, be2_ref, out_ref):
    dd = w1_ref.shape[0]
    oo = w2_ref.shape[0]
    kk = mv_ref.shape[0]
    hh = wo_ref.shape[1] // mv_ref.shape[1]

    vmean = jnp.sum(mv_ref[...], axis=0, keepdims=True) / kk
    c_hv = jnp.concatenate([vmean] * hh, axis=1)
    c_vec = jax.lax.dot_general(
        c_hv, wo_ref[...], dimension_numbers=(((1,), (1,)), ((), ())),
        preferred_element_type=jnp.float32)
    w2g = w2_ref[...] * g1_ref[...]
    s_row = jax.lax.dot_general(
        g1_ref[...], w2_ref[...], dimension_numbers=(((1,), (1,)), ((), ())),
        preferred_element_type=jnp.float32)
    t_row = jax.lax.dot_general(
        be1_ref[...], w2_ref[...], dimension_numbers=(((1,), (1,)), ((), ())),
        preferred_element_type=jnp.float32) + b2_ref[...]
    cp_row = jax.lax.dot_general(
        c_vec, w2g, dimension_numbers=(((1,), (1,)), ((), ())),
        preferred_element_type=jnp.float32)
    sum_c = jnp.sum(c_vec, axis=1, keepdims=True)
    sum_c2 = jnp.sum(c_vec * c_vec, axis=1, keepdims=True)
    waug = jnp.concatenate(
        [w2g, jnp.ones((1, dd), jnp.float32), c_vec,
         jnp.zeros((_PAD - oo - 2, dd), jnp.float32)], axis=0)

    h = jax.lax.dot_general(
        x_ref[...], w1_ref[...],
        dimension_numbers=(((1,), (1,)), ((), ())),
        preferred_element_type=jnp.float32)
    h = jnp.maximum(h + b1_ref[...], 0.0)
    hsq = jnp.sum(h * h, axis=1, keepdims=True)
    p = jax.lax.dot_general(
        h, waug, dimension_numbers=(((1,), (1,)), ((), ())),
        preferred_element_type=jnp.float32)

    mean1 = (p[:, oo:oo + 1] + sum_c) / dd
    e2 = (hsq + 2.0 * p[:, oo + 1:oo + 2] + sum_c2) / dd
    var1 = e2 - mean1 * mean1
    rstd1 = 1.0 / jnp.sqrt(var1 + 1e-5)
    x2 = (p[:, 0:oo] + cp_row - mean1 * s_row) * rstd1 + t_row

    mean2 = jnp.mean(x2, axis=1, keepdims=True)
    cen2 = x2 - mean2
    var2 = jnp.mean(cen2 * cen2, axis=1, keepdims=True)
    y = cen2 / jnp.sqrt(var2 + 1e-5) * g2_ref[...] + be2_ref[...]
    out_ref[...] = jax.nn.sigmoid(y)


def kernel(X, W1, b1, mem_keys, mem_values, Wq, Wo, ln1_g, ln1_b,
           W2, b2, ln2_g, ln2_b):
    del mem_keys, Wq
    B, D = X.shape
    O = W2.shape[0]
    grid = (B // _B_BLK,)

    def rows(i):
        return (i, 0)

    def whole(i):
        return (0, 0)

    return pl.pallas_call(
        _fused_kernel,
        grid=grid,
        in_specs=[
            pl.BlockSpec((_B_BLK, D), rows),
            pl.BlockSpec((D, D), whole),
            pl.BlockSpec((1, D), whole),
            pl.BlockSpec(mem_values.shape, whole),
            pl.BlockSpec(Wo.shape, whole),
            pl.BlockSpec((1, D), whole),
            pl.BlockSpec((1, D), whole),
            pl.BlockSpec(W2.shape, whole),
            pl.BlockSpec((1, O), whole),
            pl.BlockSpec((1, O), whole),
            pl.BlockSpec((1, O), whole),
        ],
        out_specs=pl.BlockSpec((_B_BLK, O), rows),
        out_shape=jax.ShapeDtypeStruct((B, O), jnp.float32),
    )(X, W1, b1.reshape(1, D), mem_values, Wo,
      ln1_g.reshape(1, D), ln1_b.reshape(1, D), W2,
      b2.reshape(1, O), ln2_g.reshape(1, O), ln2_b.reshape(1, O))


# speedup vs baseline: 1.3631x; 1.0084x over previous
import jax
import jax.numpy as jnp
from jax.experimental import pallas as pl

_B_BLK = 1024
_PAD = 128


def _fused_kernel(x_ref, w1_ref, b1_ref, mv_ref, wo_ref, g1_ref, be1_ref,
                  w2_ref, b2_ref, g2_ref, be2_ref, out_ref):
    dd = w1_ref.shape[0]
    oo = w2_ref.shape[0]
    kk = mv_ref.shape[0]
    hh = wo_ref.shape[1] // mv_ref.shape[1]

    vmean = jnp.sum(mv_ref[...], axis=0, keepdims=True) / kk
    c_hv = jnp.concatenate([vmean] * hh, axis=1)
    c_vec = jax.lax.dot_general(
        c_hv, wo_ref[...], dimension_numbers=(((1,), (1,)), ((), ())),
        preferred_element_type=jnp.float32)
    w2g = w2_ref[...] * g1_ref[...]
    s_row = jax.lax.dot_general(
        g1_ref[...], w2_ref[...], dimension_numbers=(((1,), (1,)), ((), ())),
        preferred_element_type=jnp.float32)
    t_row = jax.lax.dot_general(
        be1_ref[...], w2_ref[...], dimension_numbers=(((1,), (1,)), ((), ())),
        preferred_element_type=jnp.float32) + b2_ref[...]
    cp_row = jax.lax.dot_general(
        c_vec, w2g, dimension_numbers=(((1,), (1,)), ((), ())),
        preferred_element_type=jnp.float32)
    sum_c = jnp.sum(c_vec, axis=1, keepdims=True)
    sum_c2 = jnp.sum(c_vec * c_vec, axis=1, keepdims=True)
    waug = jnp.concatenate(
        [w2g, jnp.ones((1, dd), jnp.float32), c_vec,
         jnp.zeros((_PAD - oo - 2, dd), jnp.float32)], axis=0)

    h = jax.lax.dot_general(
        x_ref[...], w1_ref[...],
        dimension_numbers=(((1,), (1,)), ((), ())),
        preferred_element_type=jnp.float32)
    h = jnp.maximum(h + b1_ref[...], 0.0)
    hsq = jnp.sum(h * h, axis=1, keepdims=True)
    p = jax.lax.dot_general(
        h, waug, dimension_numbers=(((1,), (1,)), ((), ())),
        preferred_element_type=jnp.float32)

    mean1 = (p[:, oo:oo + 1] + sum_c) / dd
    e2 = (hsq + 2.0 * p[:, oo + 1:oo + 2] + sum_c2) / dd
    var1 = e2 - mean1 * mean1
    rstd1 = 1.0 / jnp.sqrt(var1 + 1e-5)
    x2 = (p[:, 0:oo] + cp_row - mean1 * s_row) * rstd1 + t_row

    mean2 = jnp.mean(x2, axis=1, keepdims=True)
    cen2 = x2 - mean2
    var2 = jnp.mean(cen2 * cen2, axis=1, keepdims=True)
    y = cen2 / jnp.sqrt(var2 + 1e-5) * g2_ref[...] + be2_ref[...]
    out_ref[...] = jax.nn.sigmoid(y)


def kernel(X, W1, b1, mem_keys, mem_values, Wq, Wo, ln1_g, ln1_b,
           W2, b2, ln2_g, ln2_b):
    del mem_keys, Wq
    B, D = X.shape
    O = W2.shape[0]
    grid = (B // _B_BLK,)

    def rows(i):
        return (i, 0)

    def whole(i):
        return (0, 0)

    return pl.pallas_call(
        _fused_kernel,
        grid=grid,
        in_specs=[
            pl.BlockSpec((_B_BLK, D), rows),
            pl.BlockSpec((D, D), whole),
            pl.BlockSpec((1, D), whole),
            pl.BlockSpec(mem_values.shape, whole),
            pl.BlockSpec(Wo.shape, whole),
            pl.BlockSpec((1, D), whole),
            pl.BlockSpec((1, D), whole),
            pl.BlockSpec(W2.shape, whole),
            pl.BlockSpec((1, O), whole),
            pl.BlockSpec((1, O), whole),
            pl.BlockSpec((1, O), whole),
        ],
        out_specs=pl.BlockSpec((_B_BLK, O), rows),
        out_shape=jax.ShapeDtypeStruct((B, O), jnp.float32),
    )(X, W1, b1.reshape(1, D), mem_values, Wo,
      ln1_g.reshape(1, D), ln1_b.reshape(1, D), W2,
      b2.reshape(1, O), ln2_g.reshape(1, O), ln2_b.reshape(1, O))
